# baseline (speedup 1.0000x reference)
import jax
import jax.numpy as jnp
from jax.experimental import pallas as pl
from jax.experimental.pallas import tpu as pltpu

_ROWS = 1024
_COLS = 100000
_RANK = 100
_BLK = 8
_CW = 128
_NCHUNK = 782
_PADW = _NCHUNK * _CW
_NEG = float("-inf")


def _body(x_ref, *rest):
    xr = rest[:_BLK]
    lab_ref, vals_ref, pos_ref = rest[_BLK:]

    lane = jax.lax.broadcasted_iota(jnp.int32, (_BLK, _CW), 1)
    lanec = jax.lax.broadcasted_iota(jnp.int32, (_BLK, _NCHUNK), 1)
    row8 = jax.lax.broadcasted_iota(jnp.int32, (_BLK, 1), 0)
    l1 = jax.lax.broadcasted_iota(jnp.int32, (1, _CW), 1)

    def init_c(c, cmx):
        v = x_ref[:, pl.ds(c * _CW, _CW)]
        m_c = jnp.max(v, axis=1, keepdims=True)
        return jnp.where(lanec == c, m_c, cmx)

    cmx = jax.lax.fori_loop(
        jnp.int32(0), jnp.int32(_NCHUNK), init_c,
        jnp.full((_BLK, _NCHUNK), _NEG, jnp.float32))

    def extract(i, carry):
        vals_acc, idx_acc, cmx = carry
        m = jnp.max(cmx, axis=1, keepdims=True)
        c_sel = jnp.max(jnp.where(cmx == m, lanec, -1), axis=1,
                        keepdims=True)
        nm = jnp.full((_BLK, 1), _NEG, jnp.float32)
        gv = jnp.zeros((_BLK, 1), jnp.int32)
        for r in range(_BLK):
            rmask = row8 == r
            c_r = jnp.max(jnp.where(rmask, c_sel, -1))
            v = xr[r][0, :, pl.ds(c_r * _CW, _CW)]
            m_r = jnp.max(v)
            li = jnp.max(jnp.where(v == m_r, l1, -1))
            newv = jnp.where(l1 == li, _NEG, v)
            xr[r][0, :, pl.ds(c_r * _CW, _CW)] = newv
            nm = jnp.where(rmask, jnp.max(newv), nm)
            gv = jnp.where(rmask, c_r * _CW + li, gv)
        cmx = jnp.where(lanec == c_sel, nm, cmx)
        vals_acc = jnp.where(lane == (_RANK - 1) - i, m, vals_acc)
        idx_acc = jnp.where(lane == (_RANK - 1) - i, gv, idx_acc)
        return vals_acc, idx_acc, cmx

    vals_acc, idx_acc, _ = jax.lax.fori_loop(
        jnp.int32(0), jnp.int32(_RANK), extract,
        (jnp.full((_BLK, _CW), _NEG, jnp.float32),
         jnp.full((_BLK, _CW), -1, jnp.int32),
         cmx))

    labs = lab_ref[:, :]
    lw = idx_acc == labs
    pos = jnp.max(jnp.where(lw, lane, -1), axis=1, keepdims=True)
    has = pos >= 0
    pos_ref[:, :] = jnp.where(has, pos, 0)

    lv = jnp.full((_BLK, 1), _NEG, jnp.float32)
    for r in range(_BLK):
        rmask = row8 == r
        lab_r = jnp.max(jnp.where(rmask, labs, -1))
        lc = lab_r // _CW
        lo = lab_r - lc * _CW
        v = xr[r][0, :, pl.ds(lc * _CW, _CW)]
        lv_r = jnp.max(jnp.where(l1 == lo, v, _NEG))
        lv = jnp.where(rmask, lv_r, lv)
    vals = jnp.where(jnp.logical_and(lane == 0, jnp.logical_not(has)),
                     lv, vals_acc)
    vals_ref[:, :] = vals[:, :_RANK]


def kernel(output, labels):
    x = jnp.pad(output, ((0, 0), (0, _PADW - _COLS)),
                constant_values=-jnp.inf)
    lab32 = labels.astype(jnp.int32).reshape(_ROWS, 1)
    x3 = x.reshape(_ROWS, 1, _PADW)
    imap = lambda i: (i, i * 0)
    row_specs = [
        pl.BlockSpec((1, 1, _PADW), lambda i, r=r: (i * _BLK + r, i * 0, i * 0))
        for r in range(_BLK)
    ]
    vals, pos = pl.pallas_call(
        _body,
        grid=(_ROWS // _BLK,),
        in_specs=[pl.BlockSpec((_BLK, _PADW), imap)] + row_specs
        + [pl.BlockSpec((_BLK, 1), imap)],
        out_specs=[
            pl.BlockSpec((_BLK, _RANK), imap),
            pl.BlockSpec((_BLK, 1), imap),
        ],
        out_shape=[
            jax.ShapeDtypeStruct((_ROWS, _RANK), jnp.float32),
            jax.ShapeDtypeStruct((_ROWS, 1), jnp.int32),
        ],
    )(x, *([x3] * _BLK), lab32)
    return vals, pos.reshape(_ROWS).astype(labels.dtype)


# speedup vs baseline: 1.2062x; 1.2062x over previous
import jax
import jax.numpy as jnp
from jax.experimental import pallas as pl
from jax.experimental.pallas import tpu as pltpu

_ROWS = 1024
_COLS = 100000
_RANK = 100
_BLK = 8
_CW = 256
_NCHUNK = 391
_PADW = _NCHUNK * _CW
_NEG = float("-inf")


def _top3(ve, lidx, neg):
    m1 = jnp.max(ve, axis=1, keepdims=True)
    i1 = jnp.max(jnp.where(ve == m1, lidx, -1), axis=1, keepdims=True)
    ve2 = jnp.where(lidx == i1, neg, ve)
    m2 = jnp.max(ve2, axis=1, keepdims=True)
    i2 = jnp.max(jnp.where(ve2 == m2, lidx, -1), axis=1, keepdims=True)
    ve3 = jnp.where(lidx == i2, neg, ve2)
    m3 = jnp.max(ve3, axis=1, keepdims=True)
    i3 = jnp.max(jnp.where(ve3 == m3, lidx, -1), axis=1, keepdims=True)
    return m1, i1, m2, i2, m3, i3


def _body(x_ref, lab_ref, vals_ref, pos_ref):
    lane = jax.lax.broadcasted_iota(jnp.int32, (_BLK, 128), 1)
    lanec = jax.lax.broadcasted_iota(jnp.int32, (_BLK, _NCHUNK), 1)
    row8 = jax.lax.broadcasted_iota(jnp.int32, (_BLK, 1), 0)
    l1 = jax.lax.broadcasted_iota(jnp.int32, (1, _CW), 1)
    lb = jax.lax.broadcasted_iota(jnp.int32, (_BLK, _CW), 1)

    def init_c(c, state):
        M1, I1, M2, I2, M3, I3 = state
        v = x_ref[:, pl.ds(c * _CW, _CW)]
        m1, i1, m2, i2, m3, i3 = _top3(v, lb, _NEG)
        base = c * _CW
        sel = lanec == c
        return (jnp.where(sel, m1, M1), jnp.where(sel, base + i1, I1),
                jnp.where(sel, m2, M2), jnp.where(sel, base + i2, I2),
                jnp.where(sel, m3, M3), jnp.where(sel, base + i3, I3))

    zf = jnp.full((_BLK, _NCHUNK), _NEG, jnp.float32)
    zi = jnp.full((_BLK, _NCHUNK), -1, jnp.int32)
    M1, I1, M2, I2, M3, I3 = jax.lax.fori_loop(
        jnp.int32(0), jnp.int32(_NCHUNK), init_c, (zf, zi, zf, zi, zf, zi))
    D = jnp.full((_BLK, _NCHUNK), 2, jnp.int32)

    def extract(i, carry):
        vals_acc, idx_acc, M1, I1, M2, I2, M3, I3, D = carry
        m = jnp.max(M1, axis=1, keepdims=True)
        c_sel = jnp.max(jnp.where(M1 == m, lanec, -1), axis=1,
                        keepdims=True)
        win = lanec == c_sel
        widx = jnp.max(jnp.where(win, I1, -1), axis=1, keepdims=True)
        vals_acc = jnp.where(lane == (_RANK - 1) - i, m, vals_acc)
        idx_acc = jnp.where(lane == (_RANK - 1) - i, widx, idx_acc)
        min_d = jnp.min(jnp.where(win, D, 2))

        def cheap(M1, I1, M2, I2, M3, I3, D, win, c_sel, widx, m):
            return (jnp.where(win, M2, M1), jnp.where(win, I2, I1),
                    jnp.where(win, M3, M2), jnp.where(win, I3, I2),
                    jnp.where(win, _NEG, M3), jnp.where(win, -1, I3),
                    jnp.where(win, D - 1, D))

        def refill(M1, I1, M2, I2, M3, I3, D, win, c_sel, widx, m):
            for r in range(_BLK):
                rmask = row8 == r
                c_r = jnp.max(jnp.where(rmask, c_sel, -1))
                w_r = jnp.max(jnp.where(rmask, widx, -1))
                m_r = jnp.max(jnp.where(rmask, m, _NEG))
                li_r = w_r - c_r * _CW
                v = x_ref[pl.ds(r, 1), pl.ds(c_r * _CW, _CW)]
                elig = jnp.logical_or(
                    v < m_r, jnp.logical_and(v == m_r, l1 < li_r))
                ve = jnp.where(elig, v, _NEG)
                m1, i1, m2, i2, m3, i3 = _top3(ve, l1, _NEG)
                m1, i1 = jnp.max(m1), jnp.max(i1)
                m2, i2 = jnp.max(m2), jnp.max(i2)
                m3, i3 = jnp.max(m3), jnp.max(i3)
                base = c_r * _CW
                rsel = jnp.logical_and(win, rmask)
                M1 = jnp.where(rsel, m1, M1)
                I1 = jnp.where(rsel, base + i1, I1)
                M2 = jnp.where(rsel, m2, M2)
                I2 = jnp.where(rsel, base + i2, I2)
                M3 = jnp.where(rsel, m3, M3)
                I3 = jnp.where(rsel, base + i3, I3)
            return M1, I1, M2, I2, M3, I3, jnp.where(win, 2, D)

        M1, I1, M2, I2, M3, I3, D = jax.lax.cond(
            min_d > 0, cheap, refill,
            M1, I1, M2, I2, M3, I3, D, win, c_sel, widx, m)
        return vals_acc, idx_acc, M1, I1, M2, I2, M3, I3, D

    out = jax.lax.fori_loop(
        jnp.int32(0), jnp.int32(_RANK), extract,
        (jnp.full((_BLK, 128), _NEG, jnp.float32),
         jnp.full((_BLK, 128), -1, jnp.int32),
         M1, I1, M2, I2, M3, I3, D))
    vals_acc, idx_acc = out[0], out[1]

    labs = lab_ref[:, :]
    lw = idx_acc == labs
    pos = jnp.max(jnp.where(lw, lane, -1), axis=1, keepdims=True)
    has = pos >= 0
    pos_ref[:, :] = jnp.where(has, pos, 0)

    lv = jnp.full((_BLK, 1), _NEG, jnp.float32)
    for r in range(_BLK):
        rmask = row8 == r
        lab_r = jnp.max(jnp.where(rmask, labs, -1))
        lc = lab_r // _CW
        lo = lab_r - lc * _CW
        v = x_ref[pl.ds(r, 1), pl.ds(lc * _CW, _CW)]
        lv_r = jnp.max(jnp.where(l1 == lo, v, _NEG))
        lv = jnp.where(rmask, lv_r, lv)
    vals = jnp.where(jnp.logical_and(lane == 0, jnp.logical_not(has)),
                     lv, vals_acc)
    vals_ref[:, :] = vals[:, :_RANK]


def kernel(output, labels):
    x = jnp.pad(output, ((0, 0), (0, _PADW - _COLS)),
                constant_values=-jnp.inf)
    lab32 = labels.astype(jnp.int32).reshape(_ROWS, 1)
    imap = lambda i: (i, i * 0)
    vals, pos = pl.pallas_call(
        _body,
        grid=(_ROWS // _BLK,),
        in_specs=[
            pl.BlockSpec((_BLK, _PADW), imap),
            pl.BlockSpec((_BLK, 1), imap),
        ],
        out_specs=[
            pl.BlockSpec((_BLK, _RANK), imap),
            pl.BlockSpec((_BLK, 1), imap),
        ],
        out_shape=[
            jax.ShapeDtypeStruct((_ROWS, _RANK), jnp.float32),
            jax.ShapeDtypeStruct((_ROWS, 1), jnp.int32),
        ],
    )(x, lab32)
    return vals, pos.reshape(_ROWS).astype(labels.dtype)


# speedup vs baseline: 2.9315x; 2.4304x over previous
import jax
import jax.numpy as jnp
from jax.experimental import pallas as pl
from jax.experimental.pallas import tpu as pltpu

_ROWS = 1024
_COLS = 100000
_RANK = 100
_BLK = 8
_CW = 512
_NCHUNK = 196
_PADW = _NCHUNK * _CW
_NEG = float("-inf")


def _top3(ve, lidx, neg):
    m1 = jnp.max(ve, axis=1, keepdims=True)
    i1 = jnp.max(jnp.where(ve == m1, lidx, -1.0), axis=1, keepdims=True)
    ve2 = jnp.where(lidx == i1, neg, ve)
    m2 = jnp.max(ve2, axis=1, keepdims=True)
    i2 = jnp.max(jnp.where(ve2 == m2, lidx, -1.0), axis=1, keepdims=True)
    ve3 = jnp.where(lidx == i2, neg, ve2)
    m3 = jnp.max(ve3, axis=1, keepdims=True)
    i3 = jnp.max(jnp.where(ve3 == m3, lidx, -1.0), axis=1, keepdims=True)
    return m1, i1, m2, i2, m3, i3


def _body(x_ref, lab_ref, vals_ref, pos_ref):
    lane = jax.lax.broadcasted_iota(jnp.int32, (_BLK, 128), 1)
    lanec_i = jax.lax.broadcasted_iota(jnp.int32, (_BLK, _NCHUNK), 1)
    lanec = lanec_i.astype(jnp.float32)
    row8 = jax.lax.broadcasted_iota(jnp.int32, (_BLK, 1), 0)
    row8f = row8.astype(jnp.float32)
    l1 = jax.lax.broadcasted_iota(jnp.int32, (1, _CW), 1).astype(jnp.float32)
    lb = jax.lax.broadcasted_iota(jnp.int32, (_BLK, _CW), 1).astype(jnp.float32)

    def init_c(c, state):
        M1, I1, M2, I2, M3, I3 = state
        for u in range(2):
            cc = c * 2 + u
            v = x_ref[:, pl.ds(cc * _CW, _CW)]
            m1, i1, m2, i2, m3, i3 = _top3(v, lb, _NEG)
            base = (cc * _CW).astype(jnp.float32)
            sel = lanec_i == cc
            M1 = jnp.where(sel, m1, M1)
            I1 = jnp.where(sel, base + i1, I1)
            M2 = jnp.where(sel, m2, M2)
            I2 = jnp.where(sel, base + i2, I2)
            M3 = jnp.where(sel, m3, M3)
            I3 = jnp.where(sel, base + i3, I3)
        return M1, I1, M2, I2, M3, I3

    zf = jnp.full((_BLK, _NCHUNK), _NEG, jnp.float32)
    zi = jnp.full((_BLK, _NCHUNK), -1.0, jnp.float32)
    M1, I1, M2, I2, M3, I3 = jax.lax.fori_loop(
        jnp.int32(0), jnp.int32(_NCHUNK // 2), init_c,
        (zf, zi, zf, zi, zf, zi))
    D = jnp.full((_BLK, _NCHUNK), 2.0, jnp.float32)

    def extract(i, carry):
        vals_acc, idx_acc, M1, I1, M2, I2, M3, I3, D = carry
        m = jnp.max(M1, axis=1, keepdims=True)
        c_sel = jnp.max(jnp.where(M1 == m, lanec, -1.0), axis=1,
                        keepdims=True)
        win = lanec == c_sel
        widx = jnp.max(jnp.where(win, I1, -1.0), axis=1, keepdims=True)
        tgt = lane == (_RANK - 1) - i
        vals_acc = jnp.where(tgt, m, vals_acc)
        idx_acc = jnp.where(tgt, widx, idx_acc)
        min_d = jnp.min(jnp.where(win, D, 2.0))

        def cheap(M1, I1, M2, I2, M3, I3, D, win, c_sel, widx, m):
            return (jnp.where(win, M2, M1), jnp.where(win, I2, I1),
                    jnp.where(win, M3, M2), jnp.where(win, I3, I2),
                    jnp.where(win, _NEG, M3), jnp.where(win, -1.0, I3),
                    jnp.where(win, D - 1.0, D))

        def refill(M1, I1, M2, I2, M3, I3, D, win, c_sel, widx, m):
            for r in range(_BLK):
                rmask = row8f == r
                c_rf = jnp.max(jnp.where(rmask, c_sel, -1.0))
                w_r = jnp.max(jnp.where(rmask, widx, -1.0))
                m_r = jnp.max(jnp.where(rmask, m, _NEG))
                c_r = c_rf.astype(jnp.int32)
                basef = c_rf * float(_CW)
                li_r = w_r - basef
                v = x_ref[pl.ds(r, 1), pl.ds(c_r * _CW, _CW)]
                elig = jnp.logical_or(
                    v < m_r, jnp.logical_and(v == m_r, l1 < li_r))
                ve = jnp.where(elig, v, _NEG)
                m1, i1, m2, i2, m3, i3 = _top3(ve, l1, _NEG)
                m1, i1 = jnp.max(m1), jnp.max(i1)
                m2, i2 = jnp.max(m2), jnp.max(i2)
                m3, i3 = jnp.max(m3), jnp.max(i3)
                rsel = jnp.logical_and(win, rmask)
                M1 = jnp.where(rsel, m1, M1)
                I1 = jnp.where(rsel, basef + i1, I1)
                M2 = jnp.where(rsel, m2, M2)
                I2 = jnp.where(rsel, basef + i2, I2)
                M3 = jnp.where(rsel, m3, M3)
                I3 = jnp.where(rsel, basef + i3, I3)
            return M1, I1, M2, I2, M3, I3, jnp.where(win, 2.0, D)

        M1, I1, M2, I2, M3, I3, D = jax.lax.cond(
            min_d > 0.0, cheap, refill,
            M1, I1, M2, I2, M3, I3, D, win, c_sel, widx, m)
        return vals_acc, idx_acc, M1, I1, M2, I2, M3, I3, D

    out = jax.lax.fori_loop(
        jnp.int32(0), jnp.int32(_RANK), extract,
        (jnp.full((_BLK, 128), _NEG, jnp.float32),
         jnp.full((_BLK, 128), -1.0, jnp.float32),
         M1, I1, M2, I2, M3, I3, D))
    vals_acc, idx_acc = out[0], out[1]

    labs = lab_ref[:, :]
    labsf = labs.astype(jnp.float32)
    lw = idx_acc == labsf
    pos = jnp.max(jnp.where(lw, lane, -1), axis=1, keepdims=True)
    has = pos >= 0
    pos_ref[:, :] = jnp.where(has, pos, 0)

    lv = jnp.full((_BLK, 1), _NEG, jnp.float32)
    for r in range(_BLK):
        rmask = row8 == r
        lab_r = jnp.max(jnp.where(rmask, labs, -1))
        lc = lab_r // _CW
        lo = (lab_r - lc * _CW).astype(jnp.float32)
        v = x_ref[pl.ds(r, 1), pl.ds(lc * _CW, _CW)]
        lv_r = jnp.max(jnp.where(l1 == lo, v, _NEG))
        lv = jnp.where(rmask, lv_r, lv)
    vals = jnp.where(jnp.logical_and(lane == 0, jnp.logical_not(has)),
                     lv, vals_acc)
    vals_ref[:, :] = vals[:, :_RANK]


def kernel(output, labels):
    x = jnp.pad(output, ((0, 0), (0, _PADW - _COLS)),
                constant_values=-jnp.inf)
    lab32 = labels.astype(jnp.int32).reshape(_ROWS, 1)
    imap = lambda i: (i, i * 0)
    vals, pos = pl.pallas_call(
        _body,
        grid=(_ROWS // _BLK,),
        in_specs=[
            pl.BlockSpec((_BLK, _PADW), imap),
            pl.BlockSpec((_BLK, 1), imap),
        ],
        out_specs=[
            pl.BlockSpec((_BLK, _RANK), imap),
            pl.BlockSpec((_BLK, 1), imap),
        ],
        out_shape=[
            jax.ShapeDtypeStruct((_ROWS, _RANK), jnp.float32),
            jax.ShapeDtypeStruct((_ROWS, 1), jnp.int32),
        ],
    )(x, lab32)
    return vals, pos.reshape(_ROWS).astype(labels.dtype)


# speedup vs baseline: 3.1921x; 1.0889x over previous
import jax
import jax.numpy as jnp
from jax.experimental import pallas as pl
from jax.experimental.pallas import tpu as pltpu

_ROWS = 1024
_COLS = 100000
_RANK = 100
_BLK = 16
_CW = 512
_NCHUNK = 196
_PADW = _NCHUNK * _CW
_NEG = float("-inf")


def _top3(ve, lidx, neg):
    m1 = jnp.max(ve, axis=1, keepdims=True)
    i1 = jnp.max(jnp.where(ve == m1, lidx, -1.0), axis=1, keepdims=True)
    ve2 = jnp.where(lidx == i1, neg, ve)
    m2 = jnp.max(ve2, axis=1, keepdims=True)
    i2 = jnp.max(jnp.where(ve2 == m2, lidx, -1.0), axis=1, keepdims=True)
    ve3 = jnp.where(lidx == i2, neg, ve2)
    m3 = jnp.max(ve3, axis=1, keepdims=True)
    i3 = jnp.max(jnp.where(ve3 == m3, lidx, -1.0), axis=1, keepdims=True)
    return m1, i1, m2, i2, m3, i3


def _body(x_ref, lab_ref, vals_ref, pos_ref):
    lane = jax.lax.broadcasted_iota(jnp.int32, (_BLK, 128), 1)
    lanec_i = jax.lax.broadcasted_iota(jnp.int32, (_BLK, _NCHUNK), 1)
    lanec = lanec_i.astype(jnp.float32)
    row8 = jax.lax.broadcasted_iota(jnp.int32, (_BLK, 1), 0)
    row8f = row8.astype(jnp.float32)
    l1 = jax.lax.broadcasted_iota(jnp.int32, (1, _CW), 1).astype(jnp.float32)
    lb = jax.lax.broadcasted_iota(jnp.int32, (_BLK, _CW), 1).astype(jnp.float32)

    def init_c(c, state):
        M1, I1, M2, I2, M3, I3 = state
        for u in range(4):
            cc = c * 4 + u
            v = x_ref[:, pl.ds(cc * _CW, _CW)]
            m1, i1, m2, i2, m3, i3 = _top3(v, lb, _NEG)
            base = (cc * _CW).astype(jnp.float32)
            sel = lanec_i == cc
            M1 = jnp.where(sel, m1, M1)
            I1 = jnp.where(sel, base + i1, I1)
            M2 = jnp.where(sel, m2, M2)
            I2 = jnp.where(sel, base + i2, I2)
            M3 = jnp.where(sel, m3, M3)
            I3 = jnp.where(sel, base + i3, I3)
        return M1, I1, M2, I2, M3, I3

    zf = jnp.full((_BLK, _NCHUNK), _NEG, jnp.float32)
    zi = jnp.full((_BLK, _NCHUNK), -1.0, jnp.float32)
    M1, I1, M2, I2, M3, I3 = jax.lax.fori_loop(
        jnp.int32(0), jnp.int32(_NCHUNK // 4), init_c,
        (zf, zi, zf, zi, zf, zi))
    D = jnp.full((_BLK, _NCHUNK), 2.0, jnp.float32)

    def extract(i, carry):
        vals_acc, idx_acc, M1, I1, M2, I2, M3, I3, D = carry
        m = jnp.max(M1, axis=1, keepdims=True)
        c_sel = jnp.max(jnp.where(M1 == m, lanec, -1.0), axis=1,
                        keepdims=True)
        win = lanec == c_sel
        widx = jnp.max(jnp.where(win, I1, -1.0), axis=1, keepdims=True)
        tgt = lane == (_RANK - 1) - i
        vals_acc = jnp.where(tgt, m, vals_acc)
        idx_acc = jnp.where(tgt, widx, idx_acc)
        min_d = jnp.min(jnp.where(win, D, 2.0))

        def cheap(M1, I1, M2, I2, M3, I3, D, win, c_sel, widx, m):
            return (jnp.where(win, M2, M1), jnp.where(win, I2, I1),
                    jnp.where(win, M3, M2), jnp.where(win, I3, I2),
                    jnp.where(win, _NEG, M3), jnp.where(win, -1.0, I3),
                    jnp.where(win, D - 1.0, D))

        def refill(M1, I1, M2, I2, M3, I3, D, win, c_sel, widx, m):
            for r in range(_BLK):
                rmask = row8f == r
                c_rf = jnp.max(jnp.where(rmask, c_sel, -1.0))
                w_r = jnp.max(jnp.where(rmask, widx, -1.0))
                m_r = jnp.max(jnp.where(rmask, m, _NEG))
                c_r = c_rf.astype(jnp.int32)
                basef = c_rf * float(_CW)
                li_r = w_r - basef
                v = x_ref[pl.ds(r, 1), pl.ds(c_r * _CW, _CW)]
                elig = jnp.logical_or(
                    v < m_r, jnp.logical_and(v == m_r, l1 < li_r))
                ve = jnp.where(elig, v, _NEG)
                m1, i1, m2, i2, m3, i3 = _top3(ve, l1, _NEG)
                m1, i1 = jnp.max(m1), jnp.max(i1)
                m2, i2 = jnp.max(m2), jnp.max(i2)
                m3, i3 = jnp.max(m3), jnp.max(i3)
                rsel = jnp.logical_and(win, rmask)
                M1 = jnp.where(rsel, m1, M1)
                I1 = jnp.where(rsel, basef + i1, I1)
                M2 = jnp.where(rsel, m2, M2)
                I2 = jnp.where(rsel, basef + i2, I2)
                M3 = jnp.where(rsel, m3, M3)
                I3 = jnp.where(rsel, basef + i3, I3)
            return M1, I1, M2, I2, M3, I3, jnp.where(win, 2.0, D)

        M1, I1, M2, I2, M3, I3, D = jax.lax.cond(
            min_d > 0.0, cheap, refill,
            M1, I1, M2, I2, M3, I3, D, win, c_sel, widx, m)
        return vals_acc, idx_acc, M1, I1, M2, I2, M3, I3, D

    out = jax.lax.fori_loop(
        jnp.int32(0), jnp.int32(_RANK), extract,
        (jnp.full((_BLK, 128), _NEG, jnp.float32),
         jnp.full((_BLK, 128), -1.0, jnp.float32),
         M1, I1, M2, I2, M3, I3, D))
    vals_acc, idx_acc = out[0], out[1]

    labs = lab_ref[:, :]
    labsf = labs.astype(jnp.float32)
    lw = idx_acc == labsf
    pos = jnp.max(jnp.where(lw, lane, -1), axis=1, keepdims=True)
    has = pos >= 0
    pos_ref[:, :] = jnp.where(has, pos, 0)

    lv = jnp.full((_BLK, 1), _NEG, jnp.float32)
    for r in range(_BLK):
        rmask = row8 == r
        lab_r = jnp.max(jnp.where(rmask, labs, -1))
        lc = lab_r // _CW
        lo = (lab_r - lc * _CW).astype(jnp.float32)
        v = x_ref[pl.ds(r, 1), pl.ds(lc * _CW, _CW)]
        lv_r = jnp.max(jnp.where(l1 == lo, v, _NEG))
        lv = jnp.where(rmask, lv_r, lv)
    vals = jnp.where(jnp.logical_and(lane == 0, jnp.logical_not(has)),
                     lv, vals_acc)
    vals_ref[:, :] = vals[:, :_RANK]


def kernel(output, labels):
    x = jnp.pad(output, ((0, 0), (0, _PADW - _COLS)),
                constant_values=-jnp.inf)
    lab32 = labels.astype(jnp.int32).reshape(_ROWS, 1)
    imap = lambda i: (i, i * 0)
    vals, pos = pl.pallas_call(
        _body,
        grid=(_ROWS // _BLK,),
        in_specs=[
            pl.BlockSpec((_BLK, _PADW), imap),
            pl.BlockSpec((_BLK, 1), imap),
        ],
        out_specs=[
            pl.BlockSpec((_BLK, _RANK), imap),
            pl.BlockSpec((_BLK, 1), imap),
        ],
        out_shape=[
            jax.ShapeDtypeStruct((_ROWS, _RANK), jnp.float32),
            jax.ShapeDtypeStruct((_ROWS, 1), jnp.int32),
        ],
    )(x, lab32)
    return vals, pos.reshape(_ROWS).astype(labels.dtype)
